# baseline (speedup 1.0000x reference)
import functools

import jax
import jax.numpy as jnp
from jax import lax
from jax.experimental import pallas as pl
from jax.experimental.pallas import tpu as pltpu
from jax.experimental.pallas import tpu_sc as plsc

_N = 10000
_E = 320000
_C = 128
_I = 2048
_ALPHA = 0.1
_NC = 2
_NS = 16
_CH = _C // _NC
_NG = _CH // 32
_EPT = 20480
_EPAD = _EPT * _NS
_SUP = 2048
_NSUP = _EPT // _SUP
_K = 256
_NCK = _SUP // _K
_NP = 10240
_RPT = _NP // _NS
_RSUB = 160
_IPT = _I // _NS

_mesh = plsc.VectorSubcoreMesh(core_axis_name="c", subcore_axis_name="s")


@functools.partial(
    pl.kernel,
    mesh=_mesh,
    compiler_params=pltpu.CompilerParams(use_tc_tiling_on_sc=False,
                                         needs_layout_passes=False),
    out_type=jax.ShapeDtypeStruct((_NC, _I, _CH), jnp.float32),
    scratch_types=[
        pltpu.VMEM_SHARED((_NP, _CH), jnp.bfloat16),
        pltpu.VMEM_SHARED((_NP, _CH), jnp.bfloat16),
        pltpu.VMEM((_SUP,), jnp.int32),
        pltpu.VMEM((_SUP,), jnp.int32),
        pltpu.VMEM((_SUP,), jnp.float32),
        pltpu.VMEM((_K, _CH), jnp.bfloat16),
        pltpu.VMEM((_K, _CH), jnp.bfloat16),
        pltpu.VMEM((_K, _CH), jnp.bfloat16),
        pltpu.VMEM((_K, _CH), jnp.bfloat16),
        pltpu.VMEM((_RSUB, _CH), jnp.float32),
        pltpu.VMEM((_RSUB, _CH), jnp.bfloat16),
        pltpu.VMEM((_IPT,), jnp.int32),
        pltpu.SemaphoreType.DMA,
        pltpu.SemaphoreType.DMA,
        pltpu.SemaphoreType.DMA,
        pltpu.SemaphoreType.DMA,
    ],
)
def _vng_sc(p0_hbm, lpf_hbm, src_hbm, dst_hbm, val_hbm, idx_hbm, out_hbm,
            p_bf, a_bf, src_v, dst_v, val_v, gbufa, gbufb, sbufa, sbufb,
            fstg, bstg, fidx_v, ga_sem, gb_sem, sa_sem, sb_sem):
    cid = lax.axis_index("c")
    sid = lax.axis_index("s")
    rbase = sid * _RPT
    gbufs = (gbufa, gbufb)
    sbufs = (sbufa, sbufb)
    gsems = (ga_sem, gb_sem)
    ssems = (sa_sem, sb_sem)

    def _zero_bstg(r):
        z = jnp.zeros((32,), jnp.bfloat16)
        for g in range(_NG):
            bstg[r, pl.ds(g * 32, 32)] = z

    plsc.parallel_loop(0, _RSUB, 1, unroll=4)(_zero_bstg)
    for cpart in range(_RPT // _RSUB):
        r0 = rbase + cpart * _RSUB
        pltpu.sync_copy(bstg, a_bf.at[pl.ds(r0, _RSUB)])

    for cpart in range(_RPT // _RSUB):
        r0 = rbase + cpart * _RSUB
        pltpu.sync_copy(p0_hbm.at[cid, pl.ds(r0, _RSUB)], fstg)

        def _pack_init(r):
            for g in range(_NG):
                a = fstg[r, pl.ds(g * 32, 16)]
                b = fstg[r, pl.ds(g * 32 + 16, 16)]
                bstg[r, pl.ds(g * 32, 32)] = plsc.pack(
                    a, b, format=plsc.PackFormat.INTERLEAVED)

        plsc.parallel_loop(0, _RSUB, 1, unroll=2)(_pack_init)
        pltpu.sync_copy(bstg, p_bf.at[pl.ds(r0, _RSUB)])
    plsc.subcore_barrier()

    def _edge_pass(p_read, p_write):
        ebase = sid * _EPT

        def _gwait(b):
            pltpu.make_async_copy(p_read.at[src_v.at[pl.ds(0, _K)]],
                                  gbufs[b], gsems[b]).wait()

        def _swait(b):
            pltpu.make_async_copy(sbufs[b],
                                  p_write.at[dst_v.at[pl.ds(0, _K)]],
                                  ssems[b]).wait()

        def _scale_into(gbuf, sbuf, jk):
            def _scale(eb, _g=gbuf, _s=sbuf, _jk=jk):
                vv = val_v[pl.ds(_jk + eb, 16)]
                for i in range(16):
                    vb = jax.lax.broadcast(vv[i], (16,))
                    v32 = plsc.pack(vb, vb,
                                    format=plsc.PackFormat.INTERLEAVED)
                    for g in range(_NG):
                        sl = pl.ds(g * 32, 32)
                        _s[eb + i, sl] = _g[eb + i, sl] * v32

            plsc.parallel_loop(0, _K, 16, unroll=2)(_scale)

        def _super(s, carry):
            off = ebase + s * _SUP
            pltpu.sync_copy(src_hbm.at[pl.ds(off, _SUP)], src_v)
            pltpu.sync_copy(dst_hbm.at[pl.ds(off, _SUP)], dst_v)
            pltpu.sync_copy(val_hbm.at[pl.ds(off, _SUP)], val_v)

            pltpu.async_copy(p_read.at[src_v.at[pl.ds(0, _K)]],
                             gbufs[0], gsems[0])

            def _pair(p, c2):
                jk0 = (2 * p) * _K
                jk1 = jk0 + _K
                pltpu.async_copy(p_read.at[src_v.at[pl.ds(jk1, _K)]],
                                 gbufs[1], gsems[1])
                _gwait(0)

                @pl.when(p > 0)
                def _():
                    _swait(0)

                _scale_into(gbufs[0], sbufs[0], jk0)
                pltpu.async_copy(sbufs[0],
                                 p_write.at[dst_v.at[pl.ds(jk0, _K)]],
                                 ssems[0], add=True)

                @pl.when(p + 1 < _NCK // 2)
                def _():
                    pltpu.async_copy(
                        p_read.at[src_v.at[pl.ds(jk1 + _K, _K)]],
                        gbufs[0], gsems[0])

                _gwait(1)

                @pl.when(p > 0)
                def _():
                    _swait(1)

                _scale_into(gbufs[1], sbufs[1], jk1)
                pltpu.async_copy(sbufs[1],
                                 p_write.at[dst_v.at[pl.ds(jk1, _K)]],
                                 ssems[1], add=True)
                return c2

            lax.fori_loop(0, _NCK // 2, _pair, 0)
            _swait(0)
            _swait(1)
            return carry

        lax.fori_loop(0, _NSUP, _super, 0)

    def _iter(it, carry):
        _edge_pass(p_bf, a_bf)
        plsc.subcore_barrier()

        @pl.when(it == 0)
        def _():
            for cpart in range(_RPT // _RSUB):
                r0 = rbase + cpart * _RSUB
                pltpu.sync_copy(a_bf.at[pl.ds(r0, _RSUB)], bstg)
                pltpu.sync_copy(lpf_hbm.at[pl.ds(cid * _NP + r0, _RSUB)],
                                fstg)

                def _conv(r):
                    for g in range(_NG):
                        ab = bstg[r, pl.ds(g * 32, 32)]
                        a, b = plsc.unpack(
                            ab, format=plsc.PackFormat.INTERLEAVED)
                        la = fstg[r, pl.ds(g * 32, 16)]
                        lb = fstg[r, pl.ds(g * 32 + 16, 16)]
                        a = a + la * _ALPHA
                        b = b + lb * _ALPHA
                        bstg[r, pl.ds(g * 32, 32)] = plsc.pack(
                            a, b, format=plsc.PackFormat.INTERLEAVED)

                plsc.parallel_loop(0, _RSUB, 1, unroll=2)(_conv)
                pltpu.sync_copy(bstg, p_bf.at[pl.ds(r0, _RSUB)])

            def _zero2(r):
                z = jnp.zeros((32,), jnp.bfloat16)
                for g in range(_NG):
                    bstg[r, pl.ds(g * 32, 32)] = z

            plsc.parallel_loop(0, _RSUB, 1, unroll=4)(_zero2)
            for cpart in range(_RPT // _RSUB):
                r0 = rbase + cpart * _RSUB
                pltpu.sync_copy(bstg, a_bf.at[pl.ds(r0, _RSUB)])

        plsc.subcore_barrier()
        return carry

    lax.fori_loop(0, 2, _iter, 0)

    ibase = sid * _IPT
    pltpu.sync_copy(idx_hbm.at[pl.ds(ibase, _IPT)], fidx_v)
    pltpu.sync_copy(a_bf.at[fidx_v], gbufa.at[pl.ds(0, _IPT)])

    def _offs(t):
        sl = pl.ds(t, 16)
        fidx_v[sl] = fidx_v[sl] + cid * _NP

    plsc.parallel_loop(0, _IPT, 16, unroll=2)(_offs)
    pltpu.async_copy(lpf_hbm.at[fidx_v], fstg.at[pl.ds(0, _IPT)],
                     ga_sem).wait()

    def _fin(r):
        for g in range(_NG):
            ab = gbufa[r, pl.ds(g * 32, 32)]
            a, b = plsc.unpack(ab, format=plsc.PackFormat.INTERLEAVED)
            la = fstg[r, pl.ds(g * 32, 16)]
            lb = fstg[r, pl.ds(g * 32 + 16, 16)]
            fstg[r, pl.ds(g * 32, 16)] = a + la * _ALPHA
            fstg[r, pl.ds(g * 32 + 16, 16)] = b + lb * _ALPHA

    plsc.parallel_loop(0, _IPT, 1, unroll=2)(_fin)
    pltpu.sync_copy(fstg.at[pl.ds(0, _IPT)],
                    out_hbm.at[cid, pl.ds(ibase, _IPT)])


def kernel(local_preds, idx, pi_mat, edge_index, edge_vals):
    p0 = pi_mat.T.reshape(_N, _NC, _CH).transpose(1, 0, 2)
    lp = local_preds.reshape(_N, _NC, _CH).transpose(1, 0, 2)
    npad = _NP - _N
    p0 = jnp.pad(p0, ((0, 0), (0, npad), (0, 0)))
    lp = jnp.pad(lp, ((0, 0), (0, npad), (0, 0))).reshape(_NC * _NP, _CH)
    dst = edge_index[0].astype(jnp.int32)
    src = edge_index[1].astype(jnp.int32)
    pad = _EPAD - _E
    src = jnp.concatenate([src, jnp.zeros((pad,), jnp.int32)])
    dst = jnp.concatenate([dst, jnp.zeros((pad,), jnp.int32)])
    vals = jnp.concatenate([edge_vals, jnp.zeros((pad,), jnp.float32)])
    out = _vng_sc(p0, lp, src, dst, vals, idx.astype(jnp.int32))
    return jnp.concatenate([out[0], out[1]], axis=1)


# speedup vs baseline: 1.0378x; 1.0378x over previous
import functools

import jax
import jax.numpy as jnp
from jax import lax
from jax.experimental import pallas as pl
from jax.experimental.pallas import tpu as pltpu
from jax.experimental.pallas import tpu_sc as plsc

_N = 10000
_E = 320000
_C = 128
_I = 2048
_ALPHA = 0.1
_NC = 2
_NS = 16
_CH = _C // _NC
_NG = _CH // 32
_EPT = 20480
_EPAD = _EPT * _NS
_SUP = 2048
_NSUP = _EPT // _SUP
_K = 512
_NCK = _SUP // _K
_NP = 10240
_RPT = _NP // _NS
_RSUB = 160
_IPT = _I // _NS

_mesh = plsc.VectorSubcoreMesh(core_axis_name="c", subcore_axis_name="s")


@functools.partial(
    pl.kernel,
    mesh=_mesh,
    compiler_params=pltpu.CompilerParams(use_tc_tiling_on_sc=False,
                                         needs_layout_passes=False),
    out_type=jax.ShapeDtypeStruct((_NC, _I, _CH), jnp.float32),
    scratch_types=[
        pltpu.VMEM_SHARED((_NP, _CH), jnp.bfloat16),
        pltpu.VMEM_SHARED((_NP, _CH), jnp.bfloat16),
        pltpu.VMEM((_SUP,), jnp.int32),
        pltpu.VMEM((_SUP,), jnp.int32),
        pltpu.VMEM((_SUP,), jnp.float32),
        pltpu.VMEM((_K, _CH), jnp.bfloat16),
        pltpu.VMEM((_K, _CH), jnp.bfloat16),
        pltpu.VMEM((_K, _CH), jnp.bfloat16),
        pltpu.VMEM((_K, _CH), jnp.bfloat16),
        pltpu.VMEM((_RSUB, _CH), jnp.float32),
        pltpu.VMEM((_RSUB, _CH), jnp.bfloat16),
        pltpu.VMEM((_IPT,), jnp.int32),
        pltpu.SemaphoreType.DMA,
        pltpu.SemaphoreType.DMA,
        pltpu.SemaphoreType.DMA,
        pltpu.SemaphoreType.DMA,
    ],
)
def _vng_sc(p0_hbm, lpf_hbm, src_hbm, dst_hbm, val_hbm, idx_hbm, out_hbm,
            p_bf, a_bf, src_v, dst_v, val_v, gbufa, gbufb, sbufa, sbufb,
            fstg, bstg, fidx_v, ga_sem, gb_sem, sa_sem, sb_sem):
    cid = lax.axis_index("c")
    sid = lax.axis_index("s")
    rbase = sid * _RPT
    gbufs = (gbufa, gbufb)
    sbufs = (sbufa, sbufb)
    gsems = (ga_sem, gb_sem)
    ssems = (sa_sem, sb_sem)

    def _zero_bstg(r):
        z = jnp.zeros((32,), jnp.bfloat16)
        for g in range(_NG):
            bstg[r, pl.ds(g * 32, 32)] = z

    plsc.parallel_loop(0, _RSUB, 1, unroll=4)(_zero_bstg)
    for cpart in range(_RPT // _RSUB):
        r0 = rbase + cpart * _RSUB
        pltpu.sync_copy(bstg, a_bf.at[pl.ds(r0, _RSUB)])

    for cpart in range(_RPT // _RSUB):
        r0 = rbase + cpart * _RSUB
        pltpu.sync_copy(p0_hbm.at[cid, pl.ds(r0, _RSUB)], fstg)

        def _pack_init(r):
            for g in range(_NG):
                a = fstg[r, pl.ds(g * 32, 16)]
                b = fstg[r, pl.ds(g * 32 + 16, 16)]
                bstg[r, pl.ds(g * 32, 32)] = plsc.pack(
                    a, b, format=plsc.PackFormat.INTERLEAVED)

        plsc.parallel_loop(0, _RSUB, 1, unroll=2)(_pack_init)
        pltpu.sync_copy(bstg, p_bf.at[pl.ds(r0, _RSUB)])
    plsc.subcore_barrier()

    def _edge_pass(p_read, p_write):
        ebase = sid * _EPT

        def _gwait(b):
            pltpu.make_async_copy(p_read.at[src_v.at[pl.ds(0, _K)]],
                                  gbufs[b], gsems[b]).wait()

        def _swait(b):
            pltpu.make_async_copy(sbufs[b],
                                  p_write.at[dst_v.at[pl.ds(0, _K)]],
                                  ssems[b]).wait()

        def _scale_into(gbuf, sbuf, jk):
            def _scale(eb, _g=gbuf, _s=sbuf, _jk=jk):
                vv = val_v[pl.ds(_jk + eb, 16)]
                for i in range(16):
                    vb = jax.lax.broadcast(vv[i], (16,))
                    v32 = plsc.pack(vb, vb,
                                    format=plsc.PackFormat.INTERLEAVED)
                    for g in range(_NG):
                        sl = pl.ds(g * 32, 32)
                        _s[eb + i, sl] = _g[eb + i, sl] * v32

            plsc.parallel_loop(0, _K, 16, unroll=2)(_scale)

        def _super(s, carry):
            off = ebase + s * _SUP
            pltpu.sync_copy(src_hbm.at[pl.ds(off, _SUP)], src_v)
            pltpu.sync_copy(dst_hbm.at[pl.ds(off, _SUP)], dst_v)
            pltpu.sync_copy(val_hbm.at[pl.ds(off, _SUP)], val_v)

            pltpu.async_copy(p_read.at[src_v.at[pl.ds(0, _K)]],
                             gbufs[0], gsems[0])

            def _pair(p, c2):
                jk0 = (2 * p) * _K
                jk1 = jk0 + _K
                pltpu.async_copy(p_read.at[src_v.at[pl.ds(jk1, _K)]],
                                 gbufs[1], gsems[1])
                _gwait(0)

                @pl.when(p > 0)
                def _():
                    _swait(0)

                _scale_into(gbufs[0], sbufs[0], jk0)
                pltpu.async_copy(sbufs[0],
                                 p_write.at[dst_v.at[pl.ds(jk0, _K)]],
                                 ssems[0], add=True)

                @pl.when(p + 1 < _NCK // 2)
                def _():
                    pltpu.async_copy(
                        p_read.at[src_v.at[pl.ds(jk1 + _K, _K)]],
                        gbufs[0], gsems[0])

                _gwait(1)

                @pl.when(p > 0)
                def _():
                    _swait(1)

                _scale_into(gbufs[1], sbufs[1], jk1)
                pltpu.async_copy(sbufs[1],
                                 p_write.at[dst_v.at[pl.ds(jk1, _K)]],
                                 ssems[1], add=True)
                return c2

            lax.fori_loop(0, _NCK // 2, _pair, 0)
            _swait(0)
            _swait(1)
            return carry

        lax.fori_loop(0, _NSUP, _super, 0)

    def _iter(it, carry):
        _edge_pass(p_bf, a_bf)
        plsc.subcore_barrier()

        @pl.when(it == 0)
        def _():
            for cpart in range(_RPT // _RSUB):
                r0 = rbase + cpart * _RSUB
                pltpu.sync_copy(a_bf.at[pl.ds(r0, _RSUB)], bstg)
                pltpu.sync_copy(lpf_hbm.at[pl.ds(cid * _NP + r0, _RSUB)],
                                fstg)

                def _conv(r):
                    for g in range(_NG):
                        ab = bstg[r, pl.ds(g * 32, 32)]
                        a, b = plsc.unpack(
                            ab, format=plsc.PackFormat.INTERLEAVED)
                        la = fstg[r, pl.ds(g * 32, 16)]
                        lb = fstg[r, pl.ds(g * 32 + 16, 16)]
                        a = a + la * _ALPHA
                        b = b + lb * _ALPHA
                        bstg[r, pl.ds(g * 32, 32)] = plsc.pack(
                            a, b, format=plsc.PackFormat.INTERLEAVED)

                plsc.parallel_loop(0, _RSUB, 1, unroll=2)(_conv)
                pltpu.sync_copy(bstg, p_bf.at[pl.ds(r0, _RSUB)])

            def _zero2(r):
                z = jnp.zeros((32,), jnp.bfloat16)
                for g in range(_NG):
                    bstg[r, pl.ds(g * 32, 32)] = z

            plsc.parallel_loop(0, _RSUB, 1, unroll=4)(_zero2)
            for cpart in range(_RPT // _RSUB):
                r0 = rbase + cpart * _RSUB
                pltpu.sync_copy(bstg, a_bf.at[pl.ds(r0, _RSUB)])

        plsc.subcore_barrier()
        return carry

    lax.fori_loop(0, 2, _iter, 0)

    ibase = sid * _IPT
    pltpu.sync_copy(idx_hbm.at[pl.ds(ibase, _IPT)], fidx_v)
    pltpu.sync_copy(a_bf.at[fidx_v], gbufa.at[pl.ds(0, _IPT)])

    def _offs(t):
        sl = pl.ds(t, 16)
        fidx_v[sl] = fidx_v[sl] + cid * _NP

    plsc.parallel_loop(0, _IPT, 16, unroll=2)(_offs)
    pltpu.async_copy(lpf_hbm.at[fidx_v], fstg.at[pl.ds(0, _IPT)],
                     ga_sem).wait()

    def _fin(r):
        for g in range(_NG):
            ab = gbufa[r, pl.ds(g * 32, 32)]
            a, b = plsc.unpack(ab, format=plsc.PackFormat.INTERLEAVED)
            la = fstg[r, pl.ds(g * 32, 16)]
            lb = fstg[r, pl.ds(g * 32 + 16, 16)]
            fstg[r, pl.ds(g * 32, 16)] = a + la * _ALPHA
            fstg[r, pl.ds(g * 32 + 16, 16)] = b + lb * _ALPHA

    plsc.parallel_loop(0, _IPT, 1, unroll=2)(_fin)
    pltpu.sync_copy(fstg.at[pl.ds(0, _IPT)],
                    out_hbm.at[cid, pl.ds(ibase, _IPT)])


def kernel(local_preds, idx, pi_mat, edge_index, edge_vals):
    p0 = pi_mat.T.reshape(_N, _NC, _CH).transpose(1, 0, 2)
    lp = local_preds.reshape(_N, _NC, _CH).transpose(1, 0, 2)
    npad = _NP - _N
    p0 = jnp.pad(p0, ((0, 0), (0, npad), (0, 0)))
    lp = jnp.pad(lp, ((0, 0), (0, npad), (0, 0))).reshape(_NC * _NP, _CH)
    dst = edge_index[0].astype(jnp.int32)
    src = edge_index[1].astype(jnp.int32)
    pad = _EPAD - _E
    src = jnp.concatenate([src, jnp.zeros((pad,), jnp.int32)])
    dst = jnp.concatenate([dst, jnp.zeros((pad,), jnp.int32)])
    vals = jnp.concatenate([edge_vals, jnp.zeros((pad,), jnp.float32)])
    out = _vng_sc(p0, lp, src, dst, vals, idx.astype(jnp.int32))
    return jnp.concatenate([out[0], out[1]], axis=1)


# speedup vs baseline: 1.1370x; 1.0956x over previous
import functools

import jax
import jax.numpy as jnp
from jax import lax
from jax.experimental import pallas as pl
from jax.experimental.pallas import tpu as pltpu
from jax.experimental.pallas import tpu_sc as plsc

_N = 10000
_E = 320000
_C = 128
_I = 2048
_ALPHA = 0.1
_NC = 2
_NS = 16
_CH = _C // _NC
_NG = _CH // 32
_EPT = 20480
_EPAD = _EPT * _NS
_SUP = 2048
_NSUP = _EPT // _SUP
_K = 512
_NCK = _SUP // _K
_RPT = _N // _NS
_RSUB = 125
_IPT = _I // _NS

_mesh = plsc.VectorSubcoreMesh(core_axis_name="c", subcore_axis_name="s")


@functools.partial(
    pl.kernel,
    mesh=_mesh,
    compiler_params=pltpu.CompilerParams(use_tc_tiling_on_sc=False,
                                         needs_layout_passes=False),
    out_type=jax.ShapeDtypeStruct((_NC, _I, _CH), jnp.float32),
    scratch_types=[
        pltpu.VMEM_SHARED((_N, _CH), jnp.bfloat16),
        pltpu.VMEM_SHARED((_N, _CH), jnp.bfloat16),
        pltpu.VMEM((_SUP,), jnp.int32),
        pltpu.VMEM((_SUP,), jnp.int32),
        pltpu.VMEM((_SUP,), jnp.float32),
        pltpu.VMEM((_K, _CH), jnp.bfloat16),
        pltpu.VMEM((_K, _CH), jnp.bfloat16),
        pltpu.VMEM((_K, _CH), jnp.bfloat16),
        pltpu.VMEM((_K, _CH), jnp.bfloat16),
        pltpu.VMEM((_IPT, _CH), jnp.float32),
        pltpu.VMEM((_IPT, _CH), jnp.bfloat16),
        pltpu.VMEM((_IPT,), jnp.int32),
        pltpu.SemaphoreType.DMA,
        pltpu.SemaphoreType.DMA,
        pltpu.SemaphoreType.DMA,
        pltpu.SemaphoreType.DMA,
    ],
)
def _vng_sc(p0_hbm, lpf_hbm, ei_hbm, val_hbm, idx_hbm, out_hbm,
            p_bf, a_bf, src_v, dst_v, val_v, gbufa, gbufb, sbufa, sbufb,
            fstg, bstg, fidx_v, ga_sem, gb_sem, sa_sem, sb_sem):
    cid = lax.axis_index("c")
    sid = lax.axis_index("s")
    rbase = sid * _RPT
    gbufs = (gbufa, gbufb)
    sbufs = (sbufa, sbufb)
    gsems = (ga_sem, gb_sem)
    ssems = (sa_sem, sb_sem)

    def _zero_bstg(r):
        z = jnp.zeros((32,), jnp.bfloat16)
        for g in range(_NG):
            bstg[r, pl.ds(g * 32, 32)] = z

    plsc.parallel_loop(0, _RSUB, 1, unroll=4)(_zero_bstg)
    for cpart in range(_RPT // _RSUB):
        r0 = rbase + cpart * _RSUB
        pltpu.sync_copy(bstg.at[pl.ds(0, _RSUB)], a_bf.at[pl.ds(r0, _RSUB)])

    for cpart in range(_RPT // _RSUB):
        r0 = rbase + cpart * _RSUB
        pltpu.sync_copy(p0_hbm.at[cid, pl.ds(r0, _RSUB)],
                        fstg.at[pl.ds(0, _RSUB)])

        def _pack_init(r):
            for g in range(_NG):
                a = fstg[r, pl.ds(g * 32, 16)]
                b = fstg[r, pl.ds(g * 32 + 16, 16)]
                bstg[r, pl.ds(g * 32, 32)] = plsc.pack(
                    a, b, format=plsc.PackFormat.INTERLEAVED)

        plsc.parallel_loop(0, _RSUB, 1, unroll=2)(_pack_init)
        pltpu.sync_copy(bstg.at[pl.ds(0, _RSUB)], p_bf.at[pl.ds(r0, _RSUB)])
    plsc.subcore_barrier()

    def _edge_pass(p_read, p_write):
        ebase = sid * _EPT

        def _gwait(b):
            pltpu.make_async_copy(p_read.at[src_v.at[pl.ds(0, _K)]],
                                  gbufs[b], gsems[b]).wait()

        def _swait(b):
            pltpu.make_async_copy(sbufs[b],
                                  p_write.at[dst_v.at[pl.ds(0, _K)]],
                                  ssems[b]).wait()

        def _scale_into(gbuf, sbuf, jk):
            def _scale(eb, _g=gbuf, _s=sbuf, _jk=jk):
                vv = val_v[pl.ds(_jk + eb, 16)]
                for i in range(16):
                    vb = jax.lax.broadcast(vv[i], (16,))
                    v32 = plsc.pack(vb, vb,
                                    format=plsc.PackFormat.INTERLEAVED)
                    for g in range(_NG):
                        sl = pl.ds(g * 32, 32)
                        _s[eb + i, sl] = _g[eb + i, sl] * v32

            plsc.parallel_loop(0, _K, 16, unroll=2)(_scale)

        def _super(s, carry):
            off = ebase + s * _SUP
            pltpu.sync_copy(ei_hbm.at[1, pl.ds(off, _SUP)], src_v)
            pltpu.sync_copy(ei_hbm.at[0, pl.ds(off, _SUP)], dst_v)
            pltpu.sync_copy(val_hbm.at[pl.ds(off, _SUP)], val_v)

            pltpu.async_copy(p_read.at[src_v.at[pl.ds(0, _K)]],
                             gbufs[0], gsems[0])

            def _pair(p, c2):
                jk0 = (2 * p) * _K
                jk1 = jk0 + _K
                pltpu.async_copy(p_read.at[src_v.at[pl.ds(jk1, _K)]],
                                 gbufs[1], gsems[1])
                _gwait(0)

                @pl.when(p > 0)
                def _():
                    _swait(0)

                _scale_into(gbufs[0], sbufs[0], jk0)
                pltpu.async_copy(sbufs[0],
                                 p_write.at[dst_v.at[pl.ds(jk0, _K)]],
                                 ssems[0], add=True)

                @pl.when(p + 1 < _NCK // 2)
                def _():
                    pltpu.async_copy(
                        p_read.at[src_v.at[pl.ds(jk1 + _K, _K)]],
                        gbufs[0], gsems[0])

                _gwait(1)

                @pl.when(p > 0)
                def _():
                    _swait(1)

                _scale_into(gbufs[1], sbufs[1], jk1)
                pltpu.async_copy(sbufs[1],
                                 p_write.at[dst_v.at[pl.ds(jk1, _K)]],
                                 ssems[1], add=True)
                return c2

            lax.fori_loop(0, _NCK // 2, _pair, 0)
            _swait(0)
            _swait(1)
            return carry

        lax.fori_loop(0, _NSUP, _super, 0)

    def _iter(it, carry):
        _edge_pass(p_bf, a_bf)
        plsc.subcore_barrier()

        @pl.when(it == 0)
        def _():
            for cpart in range(_RPT // _RSUB):
                r0 = rbase + cpart * _RSUB
                pltpu.sync_copy(a_bf.at[pl.ds(r0, _RSUB)],
                                bstg.at[pl.ds(0, _RSUB)])
                pltpu.sync_copy(lpf_hbm.at[pl.ds(cid * _N + r0, _RSUB)],
                                fstg.at[pl.ds(0, _RSUB)])

                def _conv(r):
                    for g in range(_NG):
                        ab = bstg[r, pl.ds(g * 32, 32)]
                        a, b = plsc.unpack(
                            ab, format=plsc.PackFormat.INTERLEAVED)
                        la = fstg[r, pl.ds(g * 32, 16)]
                        lb = fstg[r, pl.ds(g * 32 + 16, 16)]
                        a = a + la * _ALPHA
                        b = b + lb * _ALPHA
                        bstg[r, pl.ds(g * 32, 32)] = plsc.pack(
                            a, b, format=plsc.PackFormat.INTERLEAVED)

                plsc.parallel_loop(0, _RSUB, 1, unroll=2)(_conv)
                pltpu.sync_copy(bstg.at[pl.ds(0, _RSUB)],
                                p_bf.at[pl.ds(r0, _RSUB)])

            def _zero2(r):
                z = jnp.zeros((32,), jnp.bfloat16)
                for g in range(_NG):
                    bstg[r, pl.ds(g * 32, 32)] = z

            plsc.parallel_loop(0, _RSUB, 1, unroll=4)(_zero2)
            for cpart in range(_RPT // _RSUB):
                r0 = rbase + cpart * _RSUB
                pltpu.sync_copy(bstg.at[pl.ds(0, _RSUB)],
                                a_bf.at[pl.ds(r0, _RSUB)])

        plsc.subcore_barrier()
        return carry

    lax.fori_loop(0, 2, _iter, 0)

    ibase = sid * _IPT
    pltpu.sync_copy(idx_hbm.at[pl.ds(ibase, _IPT)], fidx_v)
    pltpu.sync_copy(a_bf.at[fidx_v], gbufa.at[pl.ds(0, _IPT)])

    def _offs(t):
        sl = pl.ds(t, 16)
        fidx_v[sl] = fidx_v[sl] + cid * _N

    plsc.parallel_loop(0, _IPT, 16, unroll=2)(_offs)
    pltpu.async_copy(lpf_hbm.at[fidx_v], fstg.at[pl.ds(0, _IPT)],
                     ga_sem).wait()

    def _fin(r):
        for g in range(_NG):
            ab = gbufa[r, pl.ds(g * 32, 32)]
            a, b = plsc.unpack(ab, format=plsc.PackFormat.INTERLEAVED)
            la = fstg[r, pl.ds(g * 32, 16)]
            lb = fstg[r, pl.ds(g * 32 + 16, 16)]
            fstg[r, pl.ds(g * 32, 16)] = a + la * _ALPHA
            fstg[r, pl.ds(g * 32 + 16, 16)] = b + lb * _ALPHA

    plsc.parallel_loop(0, _IPT, 1, unroll=2)(_fin)
    pltpu.sync_copy(fstg.at[pl.ds(0, _IPT)],
                    out_hbm.at[cid, pl.ds(ibase, _IPT)])


def kernel(local_preds, idx, pi_mat, edge_index, edge_vals):
    p0 = pi_mat.reshape(_NC, _CH, _N).transpose(0, 2, 1)
    lp = local_preds.reshape(_N, _NC, _CH).transpose(1, 0, 2)
    lp = lp.reshape(_NC * _N, _CH)
    pad = _EPAD - _E
    ei = jnp.pad(edge_index.astype(jnp.int32), ((0, 0), (0, pad)))
    vals = jnp.pad(edge_vals, (0, pad))
    out = _vng_sc(p0, lp, ei, vals, idx.astype(jnp.int32))
    return jnp.concatenate([out[0], out[1]], axis=1)


# speedup vs baseline: 1.1644x; 1.0241x over previous
import functools

import jax
import jax.numpy as jnp
from jax import lax
from jax.experimental import pallas as pl
from jax.experimental.pallas import tpu as pltpu
from jax.experimental.pallas import tpu_sc as plsc

_N = 10000
_E = 320000
_C = 128
_I = 2048
_ALPHA = 0.1
_NC = 2
_NS = 16
_CH = _C // _NC
_NG = _CH // 32
_EPT = 20480
_EPAD = _EPT * _NS
_SUP = 2048
_NSUP = _EPT // _SUP
_K = 512
_NCK = _SUP // _K
_RPT = _N // _NS
_RSUB = 125
_IPT = _I // _NS

_mesh = plsc.VectorSubcoreMesh(core_axis_name="c", subcore_axis_name="s")


@functools.partial(
    pl.kernel,
    mesh=_mesh,
    compiler_params=pltpu.CompilerParams(use_tc_tiling_on_sc=False,
                                         needs_layout_passes=False),
    out_type=jax.ShapeDtypeStruct((_NC, _I, _CH), jnp.float32),
    scratch_types=[
        pltpu.VMEM_SHARED((_N, _CH), jnp.bfloat16),
        pltpu.VMEM_SHARED((_N, _CH), jnp.bfloat16),
        pltpu.VMEM((3, _SUP), jnp.int32),
        pltpu.VMEM((_K, _CH), jnp.bfloat16),
        pltpu.VMEM((_K, _CH), jnp.bfloat16),
        pltpu.VMEM((_K, _CH), jnp.bfloat16),
        pltpu.VMEM((_K, _CH), jnp.bfloat16),
        pltpu.VMEM((_IPT, _CH), jnp.float32),
        pltpu.VMEM((_IPT, _CH), jnp.bfloat16),
        pltpu.VMEM((_IPT,), jnp.int32),
        pltpu.SemaphoreType.DMA,
        pltpu.SemaphoreType.DMA,
        pltpu.SemaphoreType.DMA,
        pltpu.SemaphoreType.DMA,
    ],
)
def _vng_sc(p0_hbm, lpf_hbm, ev_hbm, idx_hbm, out_hbm,
            p_bf, a_bf, estg, gbufa, gbufb, sbufa, sbufb,
            fstg, bstg, fidx_v, ga_sem, gb_sem, sa_sem, sb_sem):
    cid = lax.axis_index("c")
    sid = lax.axis_index("s")
    rbase = sid * _RPT
    gbufs = (gbufa, gbufb)
    sbufs = (sbufa, sbufb)
    gsems = (ga_sem, gb_sem)
    ssems = (sa_sem, sb_sem)

    def _zero_bstg(r):
        z = jnp.zeros((32,), jnp.bfloat16)
        for g in range(_NG):
            bstg[r, pl.ds(g * 32, 32)] = z

    plsc.parallel_loop(0, _RSUB, 1, unroll=4)(_zero_bstg)
    for cpart in range(_RPT // _RSUB):
        r0 = rbase + cpart * _RSUB
        pltpu.sync_copy(bstg.at[pl.ds(0, _RSUB)], a_bf.at[pl.ds(r0, _RSUB)])

    for cpart in range(_RPT // _RSUB):
        r0 = rbase + cpart * _RSUB
        pltpu.sync_copy(p0_hbm.at[cid, pl.ds(r0, _RSUB)],
                        fstg.at[pl.ds(0, _RSUB)])

        def _pack_init(r):
            for g in range(_NG):
                a = fstg[r, pl.ds(g * 32, 16)]
                b = fstg[r, pl.ds(g * 32 + 16, 16)]
                bstg[r, pl.ds(g * 32, 32)] = plsc.pack(
                    a, b, format=plsc.PackFormat.INTERLEAVED)

        plsc.parallel_loop(0, _RSUB, 1, unroll=2)(_pack_init)
        pltpu.sync_copy(bstg.at[pl.ds(0, _RSUB)], p_bf.at[pl.ds(r0, _RSUB)])
    plsc.subcore_barrier()

    def _edge_pass(p_read, p_write):
        ebase = sid * _EPT

        def _gwait(b):
            pltpu.make_async_copy(p_read.at[estg.at[1, pl.ds(0, _K)]],
                                  gbufs[b], gsems[b]).wait()

        def _swait(b):
            pltpu.make_async_copy(sbufs[b],
                                  p_write.at[estg.at[0, pl.ds(0, _K)]],
                                  ssems[b]).wait()

        def _scale_into(gbuf, sbuf, jk):
            def _scale(eb, _g=gbuf, _s=sbuf, _jk=jk):
                vv = plsc.bitcast(estg[2, pl.ds(_jk + eb, 16)], jnp.float32)
                for i in range(16):
                    vb = jax.lax.broadcast(vv[i], (16,))
                    v32 = plsc.pack(vb, vb,
                                    format=plsc.PackFormat.INTERLEAVED)
                    for g in range(_NG):
                        sl = pl.ds(g * 32, 32)
                        _s[eb + i, sl] = _g[eb + i, sl] * v32

            plsc.parallel_loop(0, _K, 16, unroll=2)(_scale)

        def _super(s, carry):
            off = ebase + s * _SUP
            pltpu.sync_copy(ev_hbm.at[:, pl.ds(off, _SUP)], estg)

            pltpu.async_copy(p_read.at[estg.at[1, pl.ds(0, _K)]],
                             gbufs[0], gsems[0])

            def _pair(p, c2):
                jk0 = (2 * p) * _K
                jk1 = jk0 + _K
                pltpu.async_copy(p_read.at[estg.at[1, pl.ds(jk1, _K)]],
                                 gbufs[1], gsems[1])
                _gwait(0)

                @pl.when(p > 0)
                def _():
                    _swait(0)

                _scale_into(gbufs[0], sbufs[0], jk0)
                pltpu.async_copy(sbufs[0],
                                 p_write.at[estg.at[0, pl.ds(jk0, _K)]],
                                 ssems[0], add=True)

                @pl.when(p + 1 < _NCK // 2)
                def _():
                    pltpu.async_copy(
                        p_read.at[estg.at[1, pl.ds(jk1 + _K, _K)]],
                        gbufs[0], gsems[0])

                _gwait(1)

                @pl.when(p > 0)
                def _():
                    _swait(1)

                _scale_into(gbufs[1], sbufs[1], jk1)
                pltpu.async_copy(sbufs[1],
                                 p_write.at[estg.at[0, pl.ds(jk1, _K)]],
                                 ssems[1], add=True)
                return c2

            lax.fori_loop(0, _NCK // 2, _pair, 0)
            _swait(0)
            _swait(1)
            return carry

        lax.fori_loop(0, _NSUP, _super, 0)

    def _iter(it, carry):
        _edge_pass(p_bf, a_bf)
        plsc.subcore_barrier()

        @pl.when(it == 0)
        def _():
            for cpart in range(_RPT // _RSUB):
                r0 = rbase + cpart * _RSUB
                pltpu.sync_copy(a_bf.at[pl.ds(r0, _RSUB)],
                                bstg.at[pl.ds(0, _RSUB)])
                pltpu.sync_copy(lpf_hbm.at[pl.ds(cid * _N + r0, _RSUB)],
                                fstg.at[pl.ds(0, _RSUB)])

                def _conv(r):
                    for g in range(_NG):
                        ab = bstg[r, pl.ds(g * 32, 32)]
                        a, b = plsc.unpack(
                            ab, format=plsc.PackFormat.INTERLEAVED)
                        la = fstg[r, pl.ds(g * 32, 16)]
                        lb = fstg[r, pl.ds(g * 32 + 16, 16)]
                        a = a + la * _ALPHA
                        b = b + lb * _ALPHA
                        bstg[r, pl.ds(g * 32, 32)] = plsc.pack(
                            a, b, format=plsc.PackFormat.INTERLEAVED)

                plsc.parallel_loop(0, _RSUB, 1, unroll=2)(_conv)
                pltpu.sync_copy(bstg.at[pl.ds(0, _RSUB)],
                                p_bf.at[pl.ds(r0, _RSUB)])

            def _zero2(r):
                z = jnp.zeros((32,), jnp.bfloat16)
                for g in range(_NG):
                    bstg[r, pl.ds(g * 32, 32)] = z

            plsc.parallel_loop(0, _RSUB, 1, unroll=4)(_zero2)
            for cpart in range(_RPT // _RSUB):
                r0 = rbase + cpart * _RSUB
                pltpu.sync_copy(bstg.at[pl.ds(0, _RSUB)],
                                a_bf.at[pl.ds(r0, _RSUB)])

        plsc.subcore_barrier()
        return carry

    lax.fori_loop(0, 2, _iter, 0)

    ibase = sid * _IPT
    pltpu.sync_copy(idx_hbm.at[pl.ds(ibase, _IPT)], fidx_v)
    pltpu.sync_copy(a_bf.at[fidx_v], gbufa.at[pl.ds(0, _IPT)])

    def _offs(t):
        sl = pl.ds(t, 16)
        fidx_v[sl] = fidx_v[sl] + cid * _N

    plsc.parallel_loop(0, _IPT, 16, unroll=2)(_offs)
    pltpu.async_copy(lpf_hbm.at[fidx_v], fstg.at[pl.ds(0, _IPT)],
                     ga_sem).wait()

    def _fin(r):
        for g in range(_NG):
            ab = gbufa[r, pl.ds(g * 32, 32)]
            a, b = plsc.unpack(ab, format=plsc.PackFormat.INTERLEAVED)
            la = fstg[r, pl.ds(g * 32, 16)]
            lb = fstg[r, pl.ds(g * 32 + 16, 16)]
            fstg[r, pl.ds(g * 32, 16)] = a + la * _ALPHA
            fstg[r, pl.ds(g * 32 + 16, 16)] = b + lb * _ALPHA

    plsc.parallel_loop(0, _IPT, 1, unroll=2)(_fin)
    pltpu.sync_copy(fstg.at[pl.ds(0, _IPT)],
                    out_hbm.at[cid, pl.ds(ibase, _IPT)])


def kernel(local_preds, idx, pi_mat, edge_index, edge_vals):
    p0 = pi_mat.reshape(_NC, _CH, _N).transpose(0, 2, 1)
    lp = local_preds.reshape(_N, _NC, _CH).transpose(1, 0, 2)
    lp = lp.reshape(_NC * _N, _CH)
    pad = _EPAD - _E
    ev = jnp.concatenate([
        edge_index.astype(jnp.int32),
        lax.bitcast_convert_type(edge_vals, jnp.int32)[None, :]], axis=0)
    ev = jnp.pad(ev, ((0, 0), (0, pad)))
    out = _vng_sc(p0, lp, ev, idx.astype(jnp.int32))
    return jnp.concatenate([out[0], out[1]], axis=1)
